# baseline (speedup 1.0000x reference)
import jax
import jax.numpy as jnp
from jax import lax
from jax.experimental import pallas as pl
from jax.experimental.pallas import tpu as pltpu
from jax.experimental.pallas import tpu_sc as plsc

N_NODES = 10000
N_BATCH = 256
H_NODE = 128
L = 16
NW = 16
ROWS = 640
NQ = ROWS // 128
LAST_OWN = N_NODES - (NW - 1) * ROWS
LAST_BASE = N_NODES - ROWS
LAST_OFF = (NW - 1) * ROWS - LAST_BASE
LAST_NQ = LAST_OWN // 128
TAIL = LAST_OWN - LAST_NQ * 128


def _mol_mean_body(nh_hbm, bidx_hbm, out_hbm,
                   g_v, bidx2, bidx_t, ones_v, zrow,
                   tmp_s, tmp_c, blk, sh_s, sh_c, sem):
    s = lax.axis_index("s")
    lane = lax.iota(jnp.int32, L)
    izero = lane * 0
    fzero = lane.astype(jnp.float32) * 0.0
    fone = fzero + 1.0

    is_last = s == (NW - 1)
    base = jnp.where(is_last, LAST_BASE, s * ROWS).astype(jnp.int32)
    own = jnp.where(is_last, (NW - 1) * ROWS, s * ROWS).astype(jnp.int32)

    cp_g = pltpu.async_copy(
        nh_hbm.at[pl.ds(base, ROWS), pl.ds(0, L)], g_v, sem)

    for i in range(L):
        zrow[i, pl.ds(0, L)] = fzero
    for i in range(128):
        ones_v[i, pl.ds(0, L)] = fone
    for cg in range(H_NODE // L):
        for i in range(L):
            blk[i, pl.ds(cg * L, L)] = fzero

    bs = s * L
    pltpu.sync_copy(zrow, sh_s.at[pl.ds(bs, L)])
    pltpu.sync_copy(zrow, sh_c.at[pl.ds(bs, L)])

    @pl.when(jnp.logical_not(is_last))
    def _():
        cps = [pltpu.async_copy(
            bidx_hbm.at[pl.ds(own + 128 * q, 128)], bidx2.at[q], sem)
            for q in range(NQ)]
        for cp in cps:
            cp.wait()

    @pl.when(is_last)
    def _():
        cps = [pltpu.async_copy(
            bidx_hbm.at[pl.ds(own + 128 * q, 128)], bidx2.at[q], sem)
            for q in range(LAST_NQ)]
        cps.append(pltpu.async_copy(
            bidx_hbm.at[pl.ds(N_NODES - TAIL, TAIL)], bidx_t, sem))
        for cp in cps:
            cp.wait()

    plsc.subcore_barrier()
    cp_g.wait()

    @pl.when(jnp.logical_not(is_last))
    def _():
        cps = []
        for q in range(NQ):
            cps.append(pltpu.async_copy(
                g_v.at[pl.ds(q * 128, 128)], sh_s.at[bidx2.at[q]], sem,
                add=True))
            cps.append(pltpu.async_copy(
                ones_v, sh_c.at[bidx2.at[q]], sem, add=True))
        for cp in cps:
            cp.wait()

    @pl.when(is_last)
    def _():
        cps = []
        for q in range(LAST_NQ):
            cps.append(pltpu.async_copy(
                g_v.at[pl.ds(LAST_OFF + q * 128, 128)], sh_s.at[bidx2.at[q]],
                sem, add=True))
            cps.append(pltpu.async_copy(
                ones_v, sh_c.at[bidx2.at[q]], sem, add=True))
        cps.append(pltpu.async_copy(
            g_v.at[pl.ds(LAST_OFF + LAST_NQ * 128, TAIL)], sh_s.at[bidx_t],
            sem, add=True))
        cps.append(pltpu.async_copy(
            ones_v.at[pl.ds(0, TAIL)], sh_c.at[bidx_t], sem, add=True))
        for cp in cps:
            cp.wait()

    plsc.subcore_barrier()

    pltpu.sync_copy(sh_s.at[pl.ds(bs, L)], tmp_s)
    pltpu.sync_copy(sh_c.at[pl.ds(bs, L)], tmp_c)
    sums = plsc.load_gather(tmp_s, [lane, izero])
    cnts = plsc.load_gather(tmp_c, [lane, izero])
    means = sums / (cnts + 1.0)
    plsc.store_scatter(blk, [lane, izero], means)
    pltpu.sync_copy(blk, out_hbm.at[pl.ds(bs, L)])


def kernel(node_features, edge_features, edges, node_hidden, edge_hidden,
           batch_indices, W1, W2, W3, U1, U2):
    mesh = plsc.VectorSubcoreMesh(
        core_axis_name="c", subcore_axis_name="s", num_cores=1)
    f = pl.kernel(
        _mol_mean_body,
        out_type=jax.ShapeDtypeStruct((N_BATCH, H_NODE), jnp.float32),
        mesh=mesh,
        scratch_types=[
            pltpu.VMEM((ROWS, L), jnp.float32),
            pltpu.VMEM((NQ, 128), jnp.int32),
            pltpu.VMEM((TAIL,), jnp.int32),
            pltpu.VMEM((128, L), jnp.float32),
            pltpu.VMEM((L, L), jnp.float32),
            pltpu.VMEM((L, L), jnp.float32),
            pltpu.VMEM((L, L), jnp.float32),
            pltpu.VMEM((L, H_NODE), jnp.float32),
            pltpu.VMEM_SHARED((N_BATCH, L), jnp.float32),
            pltpu.VMEM_SHARED((N_BATCH, L), jnp.float32),
            pltpu.SemaphoreType.DMA,
        ],
        compiler_params=pltpu.CompilerParams(
            needs_layout_passes=False, use_tc_tiling_on_sc=False,
            skip_device_barrier=True),
    )
    return f(node_hidden, batch_indices)


# speedup vs baseline: 1.0573x; 1.0573x over previous
import jax
import jax.numpy as jnp
from jax import lax
from jax.experimental import pallas as pl
from jax.experimental.pallas import tpu as pltpu
from jax.experimental.pallas import tpu_sc as plsc

N_NODES = 10000
N_BATCH = 256
H_NODE = 128
L = 16
NW = 16
ROWS = 640
NQ = ROWS // 128
LAST_OWN = N_NODES - (NW - 1) * ROWS
LAST_BASE = N_NODES - ROWS
LAST_OFF = (NW - 1) * ROWS - LAST_BASE
LAST_NQ = LAST_OWN // 128
TAIL = LAST_OWN - LAST_NQ * 128


def _mol_mean_body(nh_hbm, bidx_hbm, out_hbm,
                   g_v, bidx2, bidx_t, zrow,
                   tmp_s, blk, sh_s, sem):
    s = lax.axis_index("s")
    lane = lax.iota(jnp.int32, L)
    izero = lane * 0
    fzero = lane.astype(jnp.float32) * 0.0
    fone = fzero + 1.0

    is_last = s == (NW - 1)
    base = jnp.where(is_last, LAST_BASE, s * ROWS).astype(jnp.int32)
    own = jnp.where(is_last, (NW - 1) * ROWS, s * ROWS).astype(jnp.int32)

    cp_g = pltpu.async_copy(
        nh_hbm.at[pl.ds(base, ROWS), pl.ds(0, L)], g_v, sem)

    for i in range(L):
        zrow[i, pl.ds(0, L)] = fzero
    for cg in range(H_NODE // L):
        for i in range(L):
            blk[i, pl.ds(cg * L, L)] = fzero

    bs = s * L
    pltpu.sync_copy(zrow, sh_s.at[pl.ds(bs, L)])

    @pl.when(jnp.logical_not(is_last))
    def _():
        cps = [pltpu.async_copy(
            bidx_hbm.at[pl.ds(own + 128 * q, 128)], bidx2.at[q], sem)
            for q in range(NQ)]
        for cp in cps:
            cp.wait()

    @pl.when(is_last)
    def _():
        cps = [pltpu.async_copy(
            bidx_hbm.at[pl.ds(own + 128 * q, 128)], bidx2.at[q], sem)
            for q in range(LAST_NQ)]
        cps.append(pltpu.async_copy(
            bidx_hbm.at[pl.ds(N_NODES - TAIL, TAIL)], bidx_t, sem))
        for cp in cps:
            cp.wait()

    plsc.subcore_barrier()
    cp_g.wait()

    ione = izero + 1
    @pl.when(jnp.logical_not(is_last))
    def _():
        for t in range(ROWS // L):
            plsc.store_scatter(g_v, [t * L + lane, ione], fone)
        cps = [pltpu.async_copy(
            g_v.at[pl.ds(q * 128, 128)], sh_s.at[bidx2.at[q]], sem, add=True)
            for q in range(NQ)]
        for cp in cps:
            cp.wait()

    @pl.when(is_last)
    def _():
        for t in range(LAST_OWN // L):
            plsc.store_scatter(g_v, [LAST_OFF + t * L + lane, ione], fone)
        cps = [pltpu.async_copy(
            g_v.at[pl.ds(LAST_OFF + q * 128, 128)], sh_s.at[bidx2.at[q]],
            sem, add=True)
            for q in range(LAST_NQ)]
        cps.append(pltpu.async_copy(
            g_v.at[pl.ds(LAST_OFF + LAST_NQ * 128, TAIL)], sh_s.at[bidx_t],
            sem, add=True))
        for cp in cps:
            cp.wait()

    plsc.subcore_barrier()

    pltpu.sync_copy(sh_s.at[pl.ds(bs, L)], tmp_s)
    sums = plsc.load_gather(tmp_s, [lane, izero])
    cnts = plsc.load_gather(tmp_s, [lane, ione])
    means = sums / (cnts + 1.0)
    plsc.store_scatter(blk, [lane, izero], means)
    pltpu.sync_copy(blk, out_hbm.at[pl.ds(bs, L)])


def kernel(node_features, edge_features, edges, node_hidden, edge_hidden,
           batch_indices, W1, W2, W3, U1, U2):
    mesh = plsc.VectorSubcoreMesh(
        core_axis_name="c", subcore_axis_name="s", num_cores=1)
    f = pl.kernel(
        _mol_mean_body,
        out_type=jax.ShapeDtypeStruct((N_BATCH, H_NODE), jnp.float32),
        mesh=mesh,
        scratch_types=[
            pltpu.VMEM((ROWS, L), jnp.float32),
            pltpu.VMEM((NQ, 128), jnp.int32),
            pltpu.VMEM((TAIL,), jnp.int32),
            pltpu.VMEM((L, L), jnp.float32),
            pltpu.VMEM((L, L), jnp.float32),
            pltpu.VMEM((L, H_NODE), jnp.float32),
            pltpu.VMEM_SHARED((N_BATCH, L), jnp.float32),
            pltpu.SemaphoreType.DMA,
        ],
        compiler_params=pltpu.CompilerParams(
            needs_layout_passes=False, use_tc_tiling_on_sc=False,
            skip_device_barrier=True),
    )
    return f(node_hidden, batch_indices)
